# baseline (speedup 1.0000x reference)
import jax
import jax.numpy as jnp
from jax import lax
from jax.experimental import pallas as pl
from jax.experimental.pallas import tpu as pltpu
from jax.experimental.pallas import tpu_sc as plsc

_SPLIT = 256
_TAIL = 128
_TBLK = 800
_G1 = 48
_G2 = 8


def _tail_table(table, dim):
    vocab = table.shape[0]

    def body(t_ref, o_ref):
        tail = t_ref[:, _SPLIT:dim]
        o_ref[...] = jnp.concatenate(
            [tail, jnp.zeros((_TBLK, _TAIL - (dim - _SPLIT)), jnp.float32)], axis=1
        )

    return pl.pallas_call(
        body,
        grid=(vocab // _TBLK,),
        in_specs=[pl.BlockSpec((_TBLK, table.shape[1]), lambda i: (i, 0))],
        out_specs=pl.BlockSpec((_TBLK, _TAIL), lambda i: (i, 0)),
        out_shape=jax.ShapeDtypeStruct((vocab, _TAIL), jnp.float32),
    )(table)


def kernel(sentence, table):
    batch, seq = sentence.shape
    vocab, dim = table.shape
    seq_p = _G1 + 2 * _G2
    sent = jnp.pad(sentence.astype(jnp.int32), ((0, 0), (0, seq_p - seq)))
    tail_tab = _tail_table(table, dim)

    info = plsc.get_sparse_core_info()
    nw = info.num_cores * info.num_subcores
    per_w = batch // nw
    assert per_w % 2 == 0
    ntail = dim - _SPLIT

    mesh = plsc.VectorSubcoreMesh(core_axis_name="core", subcore_axis_name="subcore")

    @pl.kernel(
        out_type=jax.ShapeDtypeStruct((batch, seq, dim), table.dtype),
        mesh=mesh,
        scratch_types=[
            pltpu.VMEM((seq_p,), jnp.int32),
            pltpu.VMEM((seq_p,), jnp.int32),
            pltpu.VMEM((seq, dim), jnp.float32),
            pltpu.VMEM((seq, dim), jnp.float32),
            pltpu.VMEM((_G2, dim), jnp.float32),
            pltpu.VMEM((_G2, dim), jnp.float32),
            pltpu.VMEM((_G1, _TAIL), jnp.float32),
            pltpu.VMEM((_G1, _TAIL), jnp.float32),
            pltpu.VMEM((_G2, _TAIL), jnp.float32),
            pltpu.VMEM((_G2, _TAIL), jnp.float32),
            pltpu.SemaphoreType.DMA,
            pltpu.SemaphoreType.DMA,
        ],
    )
    def gather_kernel(
        tab_hbm, tail_hbm, idx_hbm, out_hbm,
        iv_a, iv_b, rows_a, rows_b, r2_a, r2_b, tv_a, tv_b, t2_a, t2_b, sem_a, sem_b,
    ):
        wid = lax.axis_index("subcore") * info.num_cores + lax.axis_index("core")
        base = wid * per_w
        tab_main = tab_hbm.at[:, pl.ds(0, _SPLIT)]

        def issue(c, iv, rows, r2, tv, t2, sem):
            pltpu.sync_copy(idx_hbm.at[base + c], iv)
            i1 = iv.at[pl.ds(0, _G1)]
            i2 = iv.at[pl.ds(_G1, _G2)]
            return (
                pltpu.async_copy(tab_main.at[i1], rows.at[pl.ds(0, _G1), pl.ds(0, _SPLIT)], sem),
                pltpu.async_copy(tab_main.at[i2], r2.at[:, pl.ds(0, _SPLIT)], sem),
                pltpu.async_copy(tail_hbm.at[i1], tv, sem),
                pltpu.async_copy(tail_hbm.at[i2], t2, sem),
            )

        def finish(c, rows, r2, tv, t2, handles):
            for h in handles:
                h.wait()

            @pl.loop(0, _G1)
            def _(j):
                rows[j, pl.ds(_SPLIT, 16)] = tv[j, pl.ds(0, 16)]
                rows[j, pl.ds(_SPLIT + 16, 16)] = tv[j, pl.ds(16, 16)]
                rows[j, pl.ds(_SPLIT + 32, 12)] = tv[j, pl.ds(32, 12)]

            for k in range(seq - _G1):
                for c16 in range(_SPLIT // 16):
                    rows[_G1 + k, pl.ds(16 * c16, 16)] = r2[k, pl.ds(16 * c16, 16)]
                rows[_G1 + k, pl.ds(_SPLIT, 16)] = t2[k, pl.ds(0, 16)]
                rows[_G1 + k, pl.ds(_SPLIT + 16, 16)] = t2[k, pl.ds(16, 16)]
                rows[_G1 + k, pl.ds(_SPLIT + 32, 12)] = t2[k, pl.ds(32, 12)]

            pltpu.sync_copy(rows, out_hbm.at[base + c])

        @pl.loop(0, per_w, step=2)
        def _(c):
            ha = issue(c, iv_a, rows_a, r2_a, tv_a, t2_a, sem_a)
            hb = issue(c + 1, iv_b, rows_b, r2_b, tv_b, t2_b, sem_b)
            finish(c, rows_a, r2_a, tv_a, t2_a, ha)
            finish(c + 1, rows_b, r2_b, tv_b, t2_b, hb)

    return gather_kernel(table, tail_tab, sent)
